# baseline (speedup 1.0000x reference)
import jax
import jax.numpy as jnp
from jax.experimental import pallas as pl
from jax.experimental.pallas import tpu as pltpu

T, B, C, H, W = 4, 4, 192, 16, 16
E, TOPK, HID, HEADS = 8, 2, 768, 8
N = H * W
RT = B * N
R = T * RT
D = C // HEADS
QKV = 3 * C
F32 = jnp.float32


def _dott(a, b):
    return jax.lax.dot_general(a, b, (((1,), (1,)), ((), ())),
                               preferred_element_type=F32)


def _bn_scale_shift(y):
    m = jnp.sum(y, axis=0, keepdims=True) * (1.0 / R)
    msq = jnp.sum(y * y, axis=0, keepdims=True) * (1.0 / R)
    g = jax.lax.rsqrt(msq - m * m + 1e-5)
    return g, -m * g


def _lif_store(dst, y, tau, g=None, sh=None):
    r = 1.0 / tau
    c = 1.0 - r
    rows = y.shape[0] // T
    if g is None:
        theta, rho, crho = tau, 0.0, None
    else:
        a = r * g
        theta = (1.0 - sh) / a
        rho = sh / (-a)
        crho = c * rho
    vt = None
    for t in range(T):
        yt = y[t * rows:(t + 1) * rows, :]
        if t == 0:
            vt = yt if crho is None else crho + yt
        else:
            vt = c * vt + yt
        fire = vt >= theta
        dst[pl.ds(t * rows, rows), :] = jnp.where(fire, 1.0, 0.0)
        vt = jnp.where(fire, rho, vt)


def _ssa_router_kernel(x_ref, wqkv_ref, wp_ref, wr_ref,
                       h_ref, wfull_ref, x_r, s_s, qkv_s, o_s):
    for tb in range(T * B):
        x_r[pl.ds(tb * N, N), :] = jnp.transpose(
            x_ref[pl.ds(tb * C, C), :])
    x = x_r[...]
    _lif_store(s_s, x, 2.0)
    y_qkv = _dott(s_s[...], wqkv_ref[...])
    _lif_store(qkv_s, y_qkv, 2.0, *_bn_scale_shift(y_qkv))

    def tb_body(i, carry):
        base = i * N
        qkv_tb = qkv_s[pl.ds(base, N), :]
        parts = []
        for hh in range(HEADS):
            qsl = slice(hh * D, (hh + 1) * D)
            ksl = slice(C + hh * D, C + (hh + 1) * D)
            vsl = slice(2 * C + hh * D, 2 * C + (hh + 1) * D)
            a = _dott(qkv_tb[:, qsl], qkv_tb[:, ksl]) * 0.125
            parts.append(jnp.dot(a, qkv_tb[:, vsl],
                                 preferred_element_type=F32))
        o_s[pl.ds(base, N), :] = jnp.concatenate(parts, axis=1)
        return carry

    jax.lax.fori_loop(0, T * B, tb_body, 0)

    _lif_store(s_s, o_s[...], 2.0)
    yp = _dott(s_s[...], wp_ref[...])
    g, sh = _bn_scale_shift(yp)
    h = x + (yp * g + sh)
    h_ref[...] = h

    xm = 0.25 * (h[0:RT, :] + h[RT:2 * RT, :] + h[2 * RT:3 * RT, :]
                 + h[3 * RT:4 * RT, :])
    rr = _dott(xm, wr_ref[...])
    mr = jnp.sum(rr, axis=0, keepdims=True) * (1.0 / RT)
    vr = jnp.sum(rr * rr, axis=0, keepdims=True) * (1.0 / RT) - mr * mr
    gr = jax.lax.rsqrt(vr + 1e-5)
    logits = jnp.concatenate(
        [jnp.sum(rr[b * N:(b + 1) * N, :], axis=0, keepdims=True) * (1.0 / N)
         for b in range(B)], axis=0)
    logits = (logits - mr) * gr
    mx = jnp.max(logits, axis=-1, keepdims=True)
    ex = jnp.exp(logits - mx)
    probs = ex / jnp.sum(ex, axis=-1, keepdims=True)
    iota = jax.lax.broadcasted_iota(jnp.int32, (B, E), 1)
    m1 = jnp.max(probs, axis=-1, keepdims=True)
    i1 = jnp.min(jnp.where(probs == m1, iota, E), axis=-1, keepdims=True)
    oh1 = iota == i1
    pmasked = jnp.where(oh1, -1.0, probs)
    m2 = jnp.max(pmasked, axis=-1, keepdims=True)
    i2 = jnp.min(jnp.where(pmasked == m2, iota, E), axis=-1, keepdims=True)
    oh2 = iota == i2
    p1 = jnp.sum(jnp.where(oh1, probs, 0.0), axis=-1, keepdims=True)
    p2 = jnp.sum(jnp.where(oh2, probs, 0.0), axis=-1, keepdims=True)
    tot = p1 + p2
    wfull_ref[...] = jnp.where(oh1, p1 / tot, 0.0) + jnp.where(oh2, p2 / tot, 0.0)


def _experts_kernel(taus_ref, wfull_ref, h_ref, w1_ref, w2_ref, out_ref,
                    s_s, s2_s, acc_s):
    e = pl.program_id(0)

    @pl.when(e == 0)
    def _init():
        acc_s[...] = h_ref[...]

    wb = [wfull_ref[b, e] for b in range(B)]
    selected = (wb[0] > 0) | (wb[1] > 0) | (wb[2] > 0) | (wb[3] > 0)

    @pl.when(selected)
    def _compute():
        tau = taus_ref[0, e]
        _lif_store(s_s, h_ref[...], tau)
        y1 = _dott(s_s[...], w1_ref[0])
        _lif_store(s2_s, y1, tau, *_bn_scale_shift(y1))
        y2 = _dott(s2_s[...], w2_ref[0])
        g, sh = _bn_scale_shift(y2)
        for t in range(T):
            for b in range(B):
                lo = t * RT + b * N
                sl = slice(lo, lo + N)
                acc_s[sl, :] += y2[sl, :] * (wb[b] * g) + wb[b] * sh

    @pl.when(e == E - 1)
    def _emit():
        for tb in range(T * B):
            out_ref[pl.ds(tb * C, C), :] = jnp.transpose(
                acc_s[pl.ds(tb * N, N), :])


def kernel(x, Wq, Wk, Wv, Wp, Wr, W1, W2):
    x_n = x.reshape(T * B * C, N)
    taus = jnp.linspace(1.5, 4.0, E, dtype=F32).reshape(1, E)
    wqkv = jnp.concatenate([Wq, Wk, Wv], axis=0)

    h, wfull = pl.pallas_call(
        _ssa_router_kernel,
        out_shape=[jax.ShapeDtypeStruct((R, C), F32),
                   jax.ShapeDtypeStruct((B, E), F32)],
        scratch_shapes=[pltpu.VMEM((R, C), F32),
                        pltpu.VMEM((R, C), F32),
                        pltpu.VMEM((R, QKV), F32),
                        pltpu.VMEM((R, C), F32)],
    )(x_n, wqkv, Wp, Wr)

    out = pl.pallas_call(
        _experts_kernel,
        grid=(E,),
        in_specs=[
            pl.BlockSpec(memory_space=pltpu.SMEM),
            pl.BlockSpec(memory_space=pltpu.SMEM),
            pl.BlockSpec((R, C), lambda e: (0, 0)),
            pl.BlockSpec((1, HID, C), lambda e: (e, 0, 0)),
            pl.BlockSpec((1, C, HID), lambda e: (e, 0, 0)),
        ],
        out_specs=pl.BlockSpec((T * B * C, N), lambda e: (0, 0)),
        out_shape=jax.ShapeDtypeStruct((T * B * C, N), F32),
        scratch_shapes=[pltpu.VMEM((R, C), F32),
                        pltpu.VMEM((R, HID), F32),
                        pltpu.VMEM((R, C), F32)],
    )(taus, wfull, h, W1, W2)

    return out.reshape(T, B, C, H, W)


# speedup vs baseline: 1.1366x; 1.1366x over previous
import jax
import jax.numpy as jnp
from jax.experimental import pallas as pl
from jax.experimental.pallas import tpu as pltpu
from jax.experimental.pallas import tpu_sc as plsc

T, B, C, H, W = 4, 4, 192, 16, 16
E, TOPK, HID, HEADS = 8, 2, 768, 8
N = H * W
RT = B * N
R = T * RT
D = C // HEADS
QKV = 3 * C
F32 = jnp.float32


def _dott(a, b):
    return jax.lax.dot_general(a, b, (((1,), (1,)), ((), ())),
                               preferred_element_type=F32)


def _bn_scale_shift(y):
    m = jnp.sum(y, axis=0, keepdims=True) * (1.0 / R)
    msq = jnp.sum(y * y, axis=0, keepdims=True) * (1.0 / R)
    g = jax.lax.rsqrt(msq - m * m + 1e-5)
    return g, -m * g


def _lif_store(dst, y, tau, g=None, sh=None):
    r = 1.0 / tau
    c = 1.0 - r
    rows = y.shape[0] // T
    if g is None:
        theta, rho, crho = tau, 0.0, None
    else:
        a = r * g
        theta = (1.0 - sh) / a
        rho = sh / (-a)
        crho = c * rho
    vt = None
    for t in range(T):
        yt = y[t * rows:(t + 1) * rows, :]
        if t == 0:
            vt = yt if crho is None else crho + yt
        else:
            vt = c * vt + yt
        fire = vt >= theta
        dst[pl.ds(t * rows, rows), :] = jnp.where(fire, 1.0, 0.0)
        vt = jnp.where(fire, rho, vt)


def _ssa_router_kernel(x_ref, wqkv_ref, wp_ref, wr_ref,
                       h_ref, wfull_ref, s_s, qkv_s, o_s):
    x = x_ref[...]
    _lif_store(s_s, x, 2.0)
    y_qkv = _dott(s_s[...], wqkv_ref[...])
    _lif_store(qkv_s, y_qkv, 2.0, *_bn_scale_shift(y_qkv))

    def tb_body(i, carry):
        base = i * N
        qkv_tb = qkv_s[pl.ds(base, N), :]
        parts = []
        for hh in range(HEADS):
            qsl = slice(hh * D, (hh + 1) * D)
            ksl = slice(C + hh * D, C + (hh + 1) * D)
            vsl = slice(2 * C + hh * D, 2 * C + (hh + 1) * D)
            a = _dott(qkv_tb[:, qsl], qkv_tb[:, ksl]) * 0.125
            parts.append(jnp.dot(a, qkv_tb[:, vsl],
                                 preferred_element_type=F32))
        o_s[pl.ds(base, N), :] = jnp.concatenate(parts, axis=1)
        return carry

    jax.lax.fori_loop(0, T * B, tb_body, 0)

    _lif_store(s_s, o_s[...], 2.0)
    yp = _dott(s_s[...], wp_ref[...])
    g, sh = _bn_scale_shift(yp)
    h = x + (yp * g + sh)
    h_ref[...] = h

    xm = 0.25 * (h[0:RT, :] + h[RT:2 * RT, :] + h[2 * RT:3 * RT, :]
                 + h[3 * RT:4 * RT, :])
    rr = _dott(xm, wr_ref[...])
    mr = jnp.sum(rr, axis=0, keepdims=True) * (1.0 / RT)
    vr = jnp.sum(rr * rr, axis=0, keepdims=True) * (1.0 / RT) - mr * mr
    gr = jax.lax.rsqrt(vr + 1e-5)
    logits = jnp.concatenate(
        [jnp.sum(rr[b * N:(b + 1) * N, :], axis=0, keepdims=True) * (1.0 / N)
         for b in range(B)], axis=0)
    lg = (logits - mr) * gr
    top = jnp.concatenate([lg, jnp.full((B, 128 - E), -1e30, F32)], axis=1)
    wfull_ref[...] = jnp.concatenate(
        [top, jnp.full((8 - B, 128), -1e30, F32)], axis=0)


def _sc_router(logits):
    mesh = plsc.VectorSubcoreMesh(core_axis_name="c", subcore_axis_name="s")

    @pl.kernel(out_type=jax.ShapeDtypeStruct((8, 128), F32), mesh=mesh,
               scratch_types=[pltpu.VMEM((128,), F32),
                              pltpu.VMEM((128,), F32),
                              pltpu.VMEM((32,), F32),
                              pltpu.VMEM((32,), jnp.int32)])
    def rk(l_hbm, o_hbm, row_s, wrow_s, dbl_s, dbli_s):
        ic = jax.lax.axis_index("c")
        isub = jax.lax.axis_index("s")

        @pl.when((ic == 0) & (isub == 0))
        def _():
            lane = jax.lax.iota(jnp.int32, 16)

            def allmax_first(keys):
                cur, curi = keys, lane
                for stride in (1, 2, 4, 8):
                    dbl_s[pl.ds(0, 16)] = cur
                    dbl_s[pl.ds(16, 16)] = cur
                    dbli_s[pl.ds(0, 16)] = curi
                    dbli_s[pl.ds(16, 16)] = curi
                    pv = dbl_s[pl.ds(stride, 16)]
                    pi = dbli_s[pl.ds(stride, 16)]
                    take = (pv > cur) | ((pv == cur) & (pi < curi))
                    cur = jnp.where(take, pv, cur)
                    curi = jnp.where(take, pi, curi)
                return cur, curi

            for s16 in range(0, 128, 16):
                wrow_s[pl.ds(s16, 16)] = jnp.zeros((16,), F32)
            for b in range(B):
                pltpu.sync_copy(l_hbm.at[b], row_s)
                keys = row_s[pl.ds(0, 16)]
                m1, i1 = allmax_first(keys)
                keys2 = jnp.where(lane == i1, -1e30, keys)
                m2, i2 = allmax_first(keys2)
                e2 = jnp.exp(m2 - m1)
                whi = 1.0 / (1.0 + e2)
                wv = jnp.where(lane == i1, whi,
                               jnp.where(lane == i2, 1.0 - whi, 0.0))
                wrow_s[pl.ds(0, 16)] = wv
                pltpu.sync_copy(wrow_s, o_hbm.at[b])

    return rk(logits)


def _experts_kernel(taus_ref, wfull_ref, h_ref, w1_ref, w2_ref, out_ref,
                    s_s, s2_s):
    e = pl.program_id(0)

    @pl.when(e == 0)
    def _init():
        out_ref[...] = h_ref[...]

    wb = [wfull_ref[b, e] for b in range(B)]
    selected = (wb[0] > 0) | (wb[1] > 0) | (wb[2] > 0) | (wb[3] > 0)

    @pl.when(selected)
    def _compute():
        tau = taus_ref[0, e]
        _lif_store(s_s, h_ref[...], tau)
        y1 = _dott(s_s[...], w1_ref[0])
        _lif_store(s2_s, y1, tau, *_bn_scale_shift(y1))
        y2 = _dott(s2_s[...], w2_ref[0])
        g, sh = _bn_scale_shift(y2)
        for t in range(T):
            for b in range(B):
                lo = t * RT + b * N
                sl = slice(lo, lo + N)
                out_ref[sl, :] += y2[sl, :] * (wb[b] * g) + wb[b] * sh


def kernel(x, Wq, Wk, Wv, Wp, Wr, W1, W2):
    x_r = x.reshape(T, B, C, N).transpose(0, 1, 3, 2).reshape(R, C)
    taus = jnp.linspace(1.5, 4.0, E, dtype=F32).reshape(1, E)
    wqkv = jnp.concatenate([Wq, Wk, Wv], axis=0)

    h, logits = pl.pallas_call(
        _ssa_router_kernel,
        out_shape=[jax.ShapeDtypeStruct((R, C), F32),
                   jax.ShapeDtypeStruct((8, 128), F32)],
        scratch_shapes=[pltpu.VMEM((R, C), F32),
                        pltpu.VMEM((R, QKV), F32),
                        pltpu.VMEM((R, C), F32)],
    )(x_r, wqkv, Wp, Wr)

    wfull = _sc_router(logits)

    out = pl.pallas_call(
        _experts_kernel,
        grid=(E,),
        in_specs=[
            pl.BlockSpec(memory_space=pltpu.SMEM),
            pl.BlockSpec(memory_space=pltpu.SMEM),
            pl.BlockSpec((R, C), lambda e: (0, 0)),
            pl.BlockSpec((1, HID, C), lambda e: (e, 0, 0)),
            pl.BlockSpec((1, C, HID), lambda e: (e, 0, 0)),
        ],
        out_specs=pl.BlockSpec((R, C), lambda e: (0, 0)),
        out_shape=jax.ShapeDtypeStruct((R, C), F32),
        scratch_shapes=[pltpu.VMEM((R, C), F32),
                        pltpu.VMEM((R, HID), F32)],
    )(taus, wfull, h, W1, W2)

    return out.reshape(T, B, N, C).transpose(0, 1, 3, 2).reshape(T, B, C, H, W)


# speedup vs baseline: 1.1557x; 1.0168x over previous
import jax
import jax.numpy as jnp
from jax.experimental import pallas as pl
from jax.experimental.pallas import tpu as pltpu
from jax.experimental.pallas import tpu_sc as plsc

T, B, C, H, W = 4, 4, 192, 16, 16
E, TOPK, HID, HEADS = 8, 2, 768, 8
N = H * W
RT = B * N
R = T * RT
D = C // HEADS
QKV = 3 * C
F32 = jnp.float32


def _dott(a, b):
    return jax.lax.dot_general(a, b, (((1,), (1,)), ((), ())),
                               preferred_element_type=F32)


def _bn_scale_shift(y):
    m = jnp.sum(y, axis=0, keepdims=True) * (1.0 / R)
    msq = jnp.sum(y * y, axis=0, keepdims=True) * (1.0 / R)
    g = jax.lax.rsqrt(msq - m * m + 1e-5)
    return g, -m * g


def _lif_store(dst, y, tau, g=None, sh=None):
    r = 1.0 / tau
    c = 1.0 - r
    rows = y.shape[0] // T
    if g is None:
        theta, rho, crho = tau, 0.0, None
    else:
        a = r * g
        theta = (1.0 - sh) / a
        rho = sh / (-a)
        crho = c * rho
    vt = None
    for t in range(T):
        yt = y[t * rows:(t + 1) * rows, :]
        if t == 0:
            vt = yt if crho is None else crho + yt
        else:
            vt = c * vt + yt
        fire = vt >= theta
        dst[pl.ds(t * rows, rows), :] = jnp.where(fire, 1.0, 0.0)
        vt = jnp.where(fire, rho, vt)


def _ssa_router_kernel(x_ref, wq_ref, wk_ref, wv_ref, wp_ref, wr_ref,
                       h_ref, wfull_ref, s_s, qkv_s, o_s):
    x = x_ref[...]
    _lif_store(s_s, x, 2.0)
    s = s_s[...]
    y_qkv = jnp.concatenate(
        [_dott(s, wq_ref[...]), _dott(s, wk_ref[...]),
         _dott(s, wv_ref[...])], axis=1)
    _lif_store(qkv_s, y_qkv, 2.0, *_bn_scale_shift(y_qkv))

    def tb_body(i, carry):
        base = i * N
        qkv_tb = qkv_s[pl.ds(base, N), :]
        parts = []
        for hh in range(HEADS):
            qsl = slice(hh * D, (hh + 1) * D)
            ksl = slice(C + hh * D, C + (hh + 1) * D)
            vsl = slice(2 * C + hh * D, 2 * C + (hh + 1) * D)
            a = _dott(qkv_tb[:, qsl], qkv_tb[:, ksl]) * 0.125
            parts.append(jnp.dot(a, qkv_tb[:, vsl],
                                 preferred_element_type=F32))
        o_s[pl.ds(base, N), :] = jnp.concatenate(parts, axis=1)
        return carry

    jax.lax.fori_loop(0, T * B, tb_body, 0)

    _lif_store(s_s, o_s[...], 2.0)
    yp = _dott(s_s[...], wp_ref[...])
    g, sh = _bn_scale_shift(yp)
    h = x + (yp * g + sh)
    h_ref[...] = h

    xm = 0.25 * (h[0:RT, :] + h[RT:2 * RT, :] + h[2 * RT:3 * RT, :]
                 + h[3 * RT:4 * RT, :])
    rr = _dott(xm, wr_ref[...])
    mr = jnp.sum(rr, axis=0, keepdims=True) * (1.0 / RT)
    vr = jnp.sum(rr * rr, axis=0, keepdims=True) * (1.0 / RT) - mr * mr
    gr = jax.lax.rsqrt(vr + 1e-5)
    logits = jnp.concatenate(
        [jnp.sum(rr[b * N:(b + 1) * N, :], axis=0, keepdims=True) * (1.0 / N)
         for b in range(B)], axis=0)
    lg = (logits - mr) * gr
    top = jnp.concatenate([lg, jnp.full((B, 128 - E), -1e30, F32)], axis=1)
    wfull_ref[...] = jnp.concatenate(
        [top, jnp.full((8 - B, 128), -1e30, F32)], axis=0)


def _sc_router(logits):
    mesh = plsc.VectorSubcoreMesh(core_axis_name="c", subcore_axis_name="s")

    @pl.kernel(out_type=jax.ShapeDtypeStruct((8, 128), F32), mesh=mesh,
               scratch_types=[pltpu.VMEM((128,), F32),
                              pltpu.VMEM((128,), F32),
                              pltpu.VMEM((32,), F32),
                              pltpu.VMEM((32,), jnp.int32)])
    def rk(l_hbm, o_hbm, row_s, wrow_s, dbl_s, dbli_s):
        ic = jax.lax.axis_index("c")
        isub = jax.lax.axis_index("s")

        @pl.when((ic == 0) & (isub == 0))
        def _():
            lane = jax.lax.iota(jnp.int32, 16)

            def allmax_first(keys):
                cur, curi = keys, lane
                for stride in (1, 2, 4, 8):
                    dbl_s[pl.ds(0, 16)] = cur
                    dbl_s[pl.ds(16, 16)] = cur
                    dbli_s[pl.ds(0, 16)] = curi
                    dbli_s[pl.ds(16, 16)] = curi
                    pv = dbl_s[pl.ds(stride, 16)]
                    pi = dbli_s[pl.ds(stride, 16)]
                    take = (pv > cur) | ((pv == cur) & (pi < curi))
                    cur = jnp.where(take, pv, cur)
                    curi = jnp.where(take, pi, curi)
                return cur, curi

            for s16 in range(0, 128, 16):
                wrow_s[pl.ds(s16, 16)] = jnp.zeros((16,), F32)
            for b in range(B):
                pltpu.sync_copy(l_hbm.at[b], row_s)
                keys = row_s[pl.ds(0, 16)]
                m1, i1 = allmax_first(keys)
                keys2 = jnp.where(lane == i1, -1e30, keys)
                m2, i2 = allmax_first(keys2)
                e2 = jnp.exp(m2 - m1)
                whi = 1.0 / (1.0 + e2)
                wv = jnp.where(lane == i1, whi,
                               jnp.where(lane == i2, 1.0 - whi, 0.0))
                wrow_s[pl.ds(0, 16)] = wv
                pltpu.sync_copy(wrow_s, o_hbm.at[b])

    return rk(logits)


def _experts_kernel(taus_ref, wfull_ref, h_ref, w1_ref, w2_ref, out_ref,
                    s_s, s2_s):
    e = pl.program_id(0)

    @pl.when(e == 0)
    def _init():
        out_ref[...] = h_ref[...]

    wb = [wfull_ref[b, e] for b in range(B)]
    selected = (wb[0] > 0) | (wb[1] > 0) | (wb[2] > 0) | (wb[3] > 0)

    @pl.when(selected)
    def _compute():
        tau = taus_ref[0, e]
        _lif_store(s_s, h_ref[...], tau)
        y1 = _dott(s_s[...], w1_ref[0])
        _lif_store(s2_s, y1, tau, *_bn_scale_shift(y1))
        y2 = _dott(s2_s[...], w2_ref[0])
        g, sh = _bn_scale_shift(y2)
        for t in range(T):
            for b in range(B):
                lo = t * RT + b * N
                sl = slice(lo, lo + N)
                out_ref[sl, :] += y2[sl, :] * (wb[b] * g) + wb[b] * sh


def kernel(x, Wq, Wk, Wv, Wp, Wr, W1, W2):
    x_r = x.reshape(T, B, C, N).transpose(0, 1, 3, 2).reshape(R, C)
    taus = jnp.linspace(1.5, 4.0, E, dtype=F32).reshape(1, E)

    h, logits = pl.pallas_call(
        _ssa_router_kernel,
        out_shape=[jax.ShapeDtypeStruct((R, C), F32),
                   jax.ShapeDtypeStruct((8, 128), F32)],
        scratch_shapes=[pltpu.VMEM((R, C), F32),
                        pltpu.VMEM((R, QKV), F32),
                        pltpu.VMEM((R, C), F32)],
    )(x_r, Wq, Wk, Wv, Wp, Wr)

    wfull = _sc_router(logits)

    out = pl.pallas_call(
        _experts_kernel,
        grid=(E,),
        in_specs=[
            pl.BlockSpec(memory_space=pltpu.SMEM),
            pl.BlockSpec(memory_space=pltpu.SMEM),
            pl.BlockSpec((R, C), lambda e: (0, 0)),
            pl.BlockSpec((1, HID, C), lambda e: (e, 0, 0)),
            pl.BlockSpec((1, C, HID), lambda e: (e, 0, 0)),
        ],
        out_specs=pl.BlockSpec((R, C), lambda e: (0, 0)),
        out_shape=jax.ShapeDtypeStruct((R, C), F32),
        scratch_shapes=[pltpu.VMEM((R, C), F32),
                        pltpu.VMEM((R, HID), F32)],
    )(taus, wfull, h, W1, W2)

    return out.reshape(T, B, N, C).transpose(0, 1, 3, 2).reshape(T, B, C, H, W)
